# baseline (speedup 1.0000x reference)
import functools

import jax
import jax.numpy as jnp
from jax import lax
from jax.experimental import pallas as pl
from jax.experimental.pallas import tpu as pltpu
from jax.experimental.pallas import tpu_sc as plsc

N = 10000
E = 320000
NC = 2
NS = 16
L = 16
NW = NC * NS
CH = 128
EPW = 10240
EPAD = NW * EPW
RPS = 624
RTAIL = N - NS * RPS
TAIL0 = NS * RPS


@functools.lru_cache(maxsize=None)
def _make_spmm(F, nch0, nch1):
  assert NS * (nch0 + nch1) == EPAD // CH and nch0 % 4 == 0 and nch1 % 4 == 0
  ncores = 1 if nch1 == 0 else NC
  mesh = plsc.VectorSubcoreMesh(core_axis_name="c", subcore_axis_name="s",
                                num_cores=ncores, num_subcores=NS)
  nmax = max(nch0, nch1)

  @functools.partial(
      pl.kernel,
      out_type=jax.ShapeDtypeStruct((ncores, N, F), jnp.float32),
      mesh=mesh,
      compiler_params=pltpu.CompilerParams(use_tc_tiling_on_sc=False),
      scratch_types=[
          pltpu.VMEM((nmax, CH), jnp.int32),
          pltpu.VMEM((nmax, CH), jnp.int32),
          pltpu.VMEM((nmax, CH), jnp.float32),
          pltpu.VMEM((RPS // 2, F), jnp.float32),
          pltpu.VMEM((CH, F), jnp.float32),
          pltpu.VMEM((CH, F), jnp.float32),
          pltpu.VMEM((CH, F), jnp.float32),
          pltpu.VMEM((CH, F), jnp.float32),
          pltpu.VMEM_SHARED((N, F), jnp.float32),
          pltpu.SemaphoreType.DMA,
          pltpu.SemaphoreType.DMA,
          pltpu.SemaphoreType.DMA,
          pltpu.SemaphoreType.DMA,
          pltpu.SemaphoreType.DMA,
          pltpu.SemaphoreType.DMA,
          pltpu.SemaphoreType.DMA,
          pltpu.SemaphoreType.DMA,
      ],
  )
  def spmm(support_hbm, src_hbm, dst_hbm, w_hbm, out_hbm,
           src_a, dst_a, w_a, zbuf, rows0, rows1, rows2, rows3, acc,
           semg0, semg1, semg2, semg3, sems0, sems1, sems2, sems3):
    c = lax.axis_index("c")
    s = lax.axis_index("s")
    nchunk = jnp.where(c == 0, nch0, nch1).astype(jnp.int32)

    @pl.when(c == 0)
    def _():
      r0 = s * nch0
      pltpu.sync_copy(src_hbm.at[pl.ds(r0, nch0)], src_a.at[pl.ds(0, nch0)])
      pltpu.sync_copy(dst_hbm.at[pl.ds(r0, nch0)], dst_a.at[pl.ds(0, nch0)])
      pltpu.sync_copy(w_hbm.at[pl.ds(r0, nch0)], w_a.at[pl.ds(0, nch0)])
    if nch1 > 0:
      @pl.when(c == 1)
      def _():
        r0 = NS * nch0 + s * nch1
        pltpu.sync_copy(src_hbm.at[pl.ds(r0, nch1)], src_a.at[pl.ds(0, nch1)])
        pltpu.sync_copy(dst_hbm.at[pl.ds(r0, nch1)], dst_a.at[pl.ds(0, nch1)])
        pltpu.sync_copy(w_hbm.at[pl.ds(r0, nch1)], w_a.at[pl.ds(0, nch1)])

    zv = jnp.zeros((L,), jnp.float32)
    def zrow(r, carry2):
      for j in range(F // L):
        zbuf[r, pl.ds(j * L, L)] = zv
      return carry2
    lax.fori_loop(0, RPS // 2, zrow, 0)
    for q in range(2):
      pltpu.sync_copy(zbuf, acc.at[pl.ds(s * RPS + q * (RPS // 2), RPS // 2)])
    @pl.when(s == NS - 1)
    def _():
      pltpu.sync_copy(zbuf.at[pl.ds(0, RTAIL)], acc.at[pl.ds(TAIL0, RTAIL)])
    plsc.subcore_barrier()

    def scale(rows_v, k):
      def grp_body(g, carry2):
        wvec = w_a[k, pl.ds(g * L, L)]
        for r in range(L):
          wspl = wvec[jnp.full((L,), r, jnp.int32)]
          row = g * L + r
          for j in range(F // L):
            sl = (row, pl.ds(j * L, L))
            rows_v[sl] = rows_v[sl] * wspl
        return carry2
      lax.fori_loop(0, CH // L, grp_body, 0)

    rows = [rows0, rows1, rows2, rows3]
    semg = [semg0, semg1, semg2, semg3]
    sems = [sems0, sems1, sems2, sems3]
    NB = 4

    for b in range(NB - 1):
      pltpu.async_copy(support_hbm.at[src_a.at[b]], rows[b], semg[b])

    def pipe_body(i, carry):
      for b in range(NB):
        k = NB * i + b
        nb = (b + NB - 1) % NB
        pltpu.make_async_copy(support_hbm.at[src_a.at[k]],
                              rows[b], semg[b]).wait()
        scale(rows[b], k)
        pltpu.async_copy(rows[b], acc.at[dst_a.at[k]], sems[b], add=True)
        if b == 0:
          @pl.when(i == 0)
          def _():
            pltpu.async_copy(support_hbm.at[src_a.at[NB - 1]],
                             rows[NB - 1], semg[NB - 1])
          @pl.when(i > 0)
          def _():
            pltpu.make_async_copy(rows[nb], acc.at[dst_a.at[k - 1]],
                                  sems[nb]).wait()
            pltpu.async_copy(support_hbm.at[src_a.at[k + NB - 1]],
                             rows[nb], semg[nb])
        else:
          @pl.when(k + NB - 1 < nchunk)
          def _():
            pltpu.make_async_copy(rows[nb], acc.at[dst_a.at[k - 1]],
                                  sems[nb]).wait()
            pltpu.async_copy(support_hbm.at[src_a.at[k + NB - 1]],
                             rows[nb], semg[nb])
      return carry

    lax.fori_loop(0, nchunk // NB, pipe_body, 0)
    for b in range(NB):
      pltpu.make_async_copy(rows[b], acc.at[dst_a.at[nchunk - NB + b]],
                            sems[b]).wait()
    plsc.subcore_barrier()
    pltpu.sync_copy(acc.at[pl.ds(s * RPS, RPS)],
                    out_hbm.at[c, pl.ds(s * RPS, RPS)])
    @pl.when(s == NS - 1)
    def _():
      pltpu.sync_copy(acc.at[pl.ds(TAIL0, RTAIL)],
                      out_hbm.at[c, pl.ds(TAIL0, RTAIL)])

  return spmm


def _mm(x, W):
  def body(x_ref, w_ref, o_ref):
    o_ref[...] = jnp.dot(x_ref[...], w_ref[...],
                         preferred_element_type=jnp.float32)
  return pl.pallas_call(
      body,
      out_shape=jax.ShapeDtypeStruct((x.shape[0], W.shape[1]), jnp.float32),
  )(x, W)


def _act(p, b):
  def body(p_ref, b_ref, o_ref):
    h = p_ref[0] + p_ref[1] + b_ref[...]
    o_ref[...] = jnp.where(h > 0, h, 0.01 * h)
  return pl.pallas_call(
      body,
      out_shape=jax.ShapeDtypeStruct((p.shape[1], p.shape[2]), jnp.float32),
  )(p, b.reshape(1, -1))


def _mm_act_mm(p, W2, b2, W3):
  def body(p_ref, w2_ref, b2_ref, w3_ref, o_ref):
    h = jnp.dot(p_ref[0] + p_ref[1], w2_ref[...],
                preferred_element_type=jnp.float32) + b2_ref[...]
    h = jnp.where(h > 0, h, 0.01 * h)
    o_ref[...] = jnp.dot(h, w3_ref[...], preferred_element_type=jnp.float32)
  return pl.pallas_call(
      body,
      out_shape=jax.ShapeDtypeStruct((p.shape[1], W3.shape[1]), jnp.float32),
  )(p, W2, b2.reshape(1, -1), W3)


def _final(p, b):
  def body(p_ref, b_ref, o_ref):
    z = p_ref[0] + b_ref[...]
    for i in range(1, p_ref.shape[0]):
      z = z + p_ref[i]
    m = jnp.max(z, axis=1, keepdims=True)
    e = jnp.exp(z - m)
    lse = jnp.log(jnp.sum(e, axis=1, keepdims=True)) + m
    o_ref[...] = z - lse
  return pl.pallas_call(
      body,
      out_shape=jax.ShapeDtypeStruct((p.shape[1], p.shape[2]), jnp.float32),
  )(p, b.reshape(1, -1))


def kernel(x, edge_index, edge_weight, W1, b1, W2, b2, W3, b3):
  pad = EPAD - E
  src = jnp.pad(edge_index[0].astype(jnp.int32), (0, pad)).reshape(-1, CH)
  dst = jnp.pad(edge_index[1].astype(jnp.int32), (0, pad)).reshape(-1, CH)
  w = jnp.pad(edge_weight, (0, pad)).reshape(-1, CH)

  W3p = jnp.pad(W3, ((0, 0), (0, 8)))
  b3p = jnp.concatenate([b3, jnp.full((8,), -1e30, jnp.float32)])

  s1 = _mm(x, W1)
  p1 = _make_spmm(16, 96, 64)(s1, src, dst, w)
  h1 = _act(p1, b1)
  p2 = _make_spmm(16, 96, 64)(h1, src, dst, w)
  s3 = _mm_act_mm(p2, W2, b2, W3p)
  p3 = _make_spmm(48, 160, 0)(s3, src, dst, w)
  return _final(p3, b3p)[:, :40]


# speedup vs baseline: 1.0914x; 1.0914x over previous
import functools

import jax
import jax.numpy as jnp
from jax import lax
from jax.experimental import pallas as pl
from jax.experimental.pallas import tpu as pltpu
from jax.experimental.pallas import tpu_sc as plsc

N = 10000
E = 320000
NC = 2
NS = 16
L = 16
NW = NC * NS
CH = 128
EPW = 10240
EPAD = NW * EPW
RPS = 624
RTAIL = N - NS * RPS
TAIL0 = NS * RPS


@functools.lru_cache(maxsize=None)
def _make_spmm(F, nch0, nch1):
  assert NS * (nch0 + nch1) == EPAD // CH and nch0 % 4 == 0 and nch1 % 4 == 0
  ncores = 1 if nch1 == 0 else NC
  mesh = plsc.VectorSubcoreMesh(core_axis_name="c", subcore_axis_name="s",
                                num_cores=ncores, num_subcores=NS)
  nmax = max(nch0, nch1)

  @functools.partial(
      pl.kernel,
      out_type=jax.ShapeDtypeStruct((ncores, N, F), jnp.float32),
      mesh=mesh,
      compiler_params=pltpu.CompilerParams(use_tc_tiling_on_sc=False),
      scratch_types=[
          pltpu.VMEM((nmax, CH), jnp.int32),
          pltpu.VMEM((nmax, CH), jnp.int32),
          pltpu.VMEM((nmax, CH), jnp.float32),
          pltpu.VMEM((RPS // 2, F), jnp.float32),
          pltpu.VMEM((CH, F), jnp.float32),
          pltpu.VMEM((CH, F), jnp.float32),
          pltpu.VMEM((CH, F), jnp.float32),
          pltpu.VMEM((CH, F), jnp.float32),
          pltpu.VMEM_SHARED((N, F), jnp.float32),
          pltpu.SemaphoreType.DMA,
          pltpu.SemaphoreType.DMA,
          pltpu.SemaphoreType.DMA,
          pltpu.SemaphoreType.DMA,
          pltpu.SemaphoreType.DMA,
          pltpu.SemaphoreType.DMA,
          pltpu.SemaphoreType.DMA,
          pltpu.SemaphoreType.DMA,
      ],
  )
  def spmm(support_hbm, src_hbm, dst_hbm, w_hbm, out_hbm,
           src_a, dst_a, w_a, zbuf, rows0, rows1, rows2, rows3, acc,
           semg0, semg1, semg2, semg3, sems0, sems1, sems2, sems3):
    c = lax.axis_index("c")
    s = lax.axis_index("s")
    nchunk = jnp.where(c == 0, nch0, nch1).astype(jnp.int32)

    @pl.when(c == 0)
    def _():
      r0 = s * nch0
      pltpu.sync_copy(src_hbm.at[pl.ds(r0, nch0)], src_a.at[pl.ds(0, nch0)])
      pltpu.sync_copy(dst_hbm.at[pl.ds(r0, nch0)], dst_a.at[pl.ds(0, nch0)])
      pltpu.sync_copy(w_hbm.at[pl.ds(r0, nch0)], w_a.at[pl.ds(0, nch0)])
    if nch1 > 0:
      @pl.when(c == 1)
      def _():
        r0 = NS * nch0 + s * nch1
        pltpu.sync_copy(src_hbm.at[pl.ds(r0, nch1)], src_a.at[pl.ds(0, nch1)])
        pltpu.sync_copy(dst_hbm.at[pl.ds(r0, nch1)], dst_a.at[pl.ds(0, nch1)])
        pltpu.sync_copy(w_hbm.at[pl.ds(r0, nch1)], w_a.at[pl.ds(0, nch1)])

    zv = jnp.zeros((L,), jnp.float32)
    def zrow(r, carry2):
      for j in range(F // L):
        zbuf[r, pl.ds(j * L, L)] = zv
      return carry2
    lax.fori_loop(0, RPS // 2, zrow, 0)
    for q in range(2):
      pltpu.sync_copy(zbuf, acc.at[pl.ds(s * RPS + q * (RPS // 2), RPS // 2)])
    @pl.when(s == NS - 1)
    def _():
      pltpu.sync_copy(zbuf.at[pl.ds(0, RTAIL)], acc.at[pl.ds(TAIL0, RTAIL)])
    plsc.subcore_barrier()

    def scale(rows_v, k):
      def grp_body(g, carry2):
        wvec = w_a[k, pl.ds(g * L, L)]
        for r in range(L):
          wspl = wvec[jnp.full((L,), r, jnp.int32)]
          row = g * L + r
          for j in range(F // L):
            sl = (row, pl.ds(j * L, L))
            rows_v[sl] = rows_v[sl] * wspl
        return carry2
      lax.fori_loop(0, CH // L, grp_body, 0)

    rows = [rows0, rows1, rows2, rows3]
    semg = [semg0, semg1, semg2, semg3]
    sems = [sems0, sems1, sems2, sems3]
    NB = 4

    for b in range(NB - 1):
      pltpu.async_copy(support_hbm.at[src_a.at[b]], rows[b], semg[b])

    def pipe_body(i, carry):
      for b in range(NB):
        k = NB * i + b
        nb = (b + NB - 1) % NB
        pltpu.make_async_copy(support_hbm.at[src_a.at[k]],
                              rows[b], semg[b]).wait()
        scale(rows[b], k)
        pltpu.async_copy(rows[b], acc.at[dst_a.at[k]], sems[b], add=True)
        if b == 0:
          @pl.when(i == 0)
          def _():
            pltpu.async_copy(support_hbm.at[src_a.at[NB - 1]],
                             rows[NB - 1], semg[NB - 1])
          @pl.when(i > 0)
          def _():
            pltpu.make_async_copy(rows[nb], acc.at[dst_a.at[k - 1]],
                                  sems[nb]).wait()
            pltpu.async_copy(support_hbm.at[src_a.at[k + NB - 1]],
                             rows[nb], semg[nb])
        else:
          @pl.when(k + NB - 1 < nchunk)
          def _():
            pltpu.make_async_copy(rows[nb], acc.at[dst_a.at[k - 1]],
                                  sems[nb]).wait()
            pltpu.async_copy(support_hbm.at[src_a.at[k + NB - 1]],
                             rows[nb], semg[nb])
      return carry

    lax.fori_loop(0, nchunk // NB, pipe_body, 0)
    for b in range(NB):
      pltpu.make_async_copy(rows[b], acc.at[dst_a.at[nchunk - NB + b]],
                            sems[b]).wait()
    plsc.subcore_barrier()
    pltpu.sync_copy(acc.at[pl.ds(s * RPS, RPS)],
                    out_hbm.at[c, pl.ds(s * RPS, RPS)])
    @pl.when(s == NS - 1)
    def _():
      pltpu.sync_copy(acc.at[pl.ds(TAIL0, RTAIL)],
                      out_hbm.at[c, pl.ds(TAIL0, RTAIL)])

  return spmm


def _mm(x, W):
  def body(x_ref, w_ref, o_ref):
    o_ref[...] = jnp.dot(x_ref[...], w_ref[...],
                         preferred_element_type=jnp.float32)
  return pl.pallas_call(
      body,
      out_shape=jax.ShapeDtypeStruct((x.shape[0], W.shape[1]), jnp.float32),
  )(x, W)


def _act(p, b):
  def body(p_ref, b_ref, o_ref):
    h = p_ref[0] + p_ref[1] + b_ref[...]
    o_ref[...] = jnp.where(h > 0, h, 0.01 * h)
  return pl.pallas_call(
      body,
      out_shape=jax.ShapeDtypeStruct((p.shape[1], p.shape[2]), jnp.float32),
  )(p, b.reshape(1, -1))


def _mm_act_mm(p, W2, b2, W3):
  def body(p_ref, w2_ref, b2_ref, w3_ref, o_ref):
    h = jnp.dot(p_ref[0] + p_ref[1], w2_ref[...],
                preferred_element_type=jnp.float32) + b2_ref[...]
    h = jnp.where(h > 0, h, 0.01 * h)
    o_ref[...] = jnp.dot(h, w3_ref[...], preferred_element_type=jnp.float32)
  return pl.pallas_call(
      body,
      out_shape=jax.ShapeDtypeStruct((p.shape[1], W3.shape[1]), jnp.float32),
  )(p, W2, b2.reshape(1, -1), W3)


def _final(p, b):
  def body(p_ref, b_ref, o_ref):
    z = p_ref[0] + b_ref[...]
    for i in range(1, p_ref.shape[0]):
      z = z + p_ref[i]
    m = jnp.max(z, axis=1, keepdims=True)
    e = jnp.exp(z - m)
    lse = jnp.log(jnp.sum(e, axis=1, keepdims=True)) + m
    o_ref[...] = z - lse
  return pl.pallas_call(
      body,
      out_shape=jax.ShapeDtypeStruct((p.shape[1], p.shape[2]), jnp.float32),
  )(p, b.reshape(1, -1))


def kernel(x, edge_index, edge_weight, W1, b1, W2, b2, W3, b3):
  pad = EPAD - E
  src = jnp.pad(edge_index[0].astype(jnp.int32), (0, pad)).reshape(-1, CH)
  dst = jnp.pad(edge_index[1].astype(jnp.int32), (0, pad)).reshape(-1, CH)
  w = jnp.pad(edge_weight, (0, pad)).reshape(-1, CH)

  W3p = jnp.pad(W3, ((0, 0), (0, 8)))
  b3p = jnp.concatenate([b3, jnp.full((8,), -1e30, jnp.float32)])

  s1 = _mm(x, W1)
  p1 = _make_spmm(16, 112, 48)(s1, src, dst, w)
  h1 = _act(p1, b1)
  p2 = _make_spmm(16, 112, 48)(h1, src, dst, w)
  s3 = _mm_act_mm(p2, W2, b2, W3p)
  p3 = _make_spmm(48, 128, 32)(s3, src, dst, w)
  return _final(p3, b3p)[:, :40]


# speedup vs baseline: 1.1149x; 1.0215x over previous
import functools

import jax
import jax.numpy as jnp
from jax import lax
from jax.experimental import pallas as pl
from jax.experimental.pallas import tpu as pltpu
from jax.experimental.pallas import tpu_sc as plsc

N = 10000
E = 320000
NC = 2
NS = 16
L = 16
NW = NC * NS
CH = 128
EPW = 10240
EPAD = NW * EPW
RPS = 624
RTAIL = N - NS * RPS
TAIL0 = NS * RPS


@functools.lru_cache(maxsize=None)
def _make_spmm(F, nch0, nch1):
  assert NS * (nch0 + nch1) == EPAD // CH and nch0 % 4 == 0 and nch1 % 4 == 0
  ncores = 1 if nch1 == 0 else NC
  mesh = plsc.VectorSubcoreMesh(core_axis_name="c", subcore_axis_name="s",
                                num_cores=ncores, num_subcores=NS)
  nmax = max(nch0, nch1)

  @functools.partial(
      pl.kernel,
      out_type=jax.ShapeDtypeStruct((ncores, N, F), jnp.float32),
      mesh=mesh,
      compiler_params=pltpu.CompilerParams(use_tc_tiling_on_sc=False),
      scratch_types=[
          pltpu.VMEM((nmax, CH), jnp.int32),
          pltpu.VMEM((nmax, CH), jnp.int32),
          pltpu.VMEM((nmax, CH), jnp.float32),
          pltpu.VMEM((RPS // 2, F), jnp.float32),
          pltpu.VMEM((CH, F), jnp.float32),
          pltpu.VMEM((CH, F), jnp.float32),
          pltpu.VMEM((CH, F), jnp.float32),
          pltpu.VMEM((CH, F), jnp.float32),
          pltpu.VMEM_SHARED((N, F), jnp.float32),
          pltpu.SemaphoreType.DMA,
          pltpu.SemaphoreType.DMA,
          pltpu.SemaphoreType.DMA,
          pltpu.SemaphoreType.DMA,
          pltpu.SemaphoreType.DMA,
          pltpu.SemaphoreType.DMA,
          pltpu.SemaphoreType.DMA,
          pltpu.SemaphoreType.DMA,
      ],
  )
  def spmm(support_hbm, src_hbm, dst_hbm, w_hbm, out_hbm,
           src_a, dst_a, w_a, zbuf, rows0, rows1, rows2, rows3, acc,
           semg0, semg1, semg2, semg3, sems0, sems1, sems2, sems3):
    c = lax.axis_index("c")
    s = lax.axis_index("s")
    nchunk = jnp.where(c == 0, nch0, nch1).astype(jnp.int32)

    @pl.when(c == 0)
    def _():
      r0 = s * nch0
      pltpu.sync_copy(src_hbm.at[pl.ds(r0, nch0)], src_a.at[pl.ds(0, nch0)])
      pltpu.sync_copy(dst_hbm.at[pl.ds(r0, nch0)], dst_a.at[pl.ds(0, nch0)])
      pltpu.sync_copy(w_hbm.at[pl.ds(r0, nch0)], w_a.at[pl.ds(0, nch0)])
    if nch1 > 0:
      @pl.when(c == 1)
      def _():
        r0 = NS * nch0 + s * nch1
        pltpu.sync_copy(src_hbm.at[pl.ds(r0, nch1)], src_a.at[pl.ds(0, nch1)])
        pltpu.sync_copy(dst_hbm.at[pl.ds(r0, nch1)], dst_a.at[pl.ds(0, nch1)])
        pltpu.sync_copy(w_hbm.at[pl.ds(r0, nch1)], w_a.at[pl.ds(0, nch1)])

    rows = [rows0, rows1, rows2, rows3]
    semg = [semg0, semg1, semg2, semg3]
    sems = [sems0, sems1, sems2, sems3]
    NB = 4

    for b in range(NB - 1):
      pltpu.async_copy(support_hbm.at[src_a.at[b]], rows[b], semg[b])

    zv = jnp.zeros((L,), jnp.float32)
    def zrow(r, carry2):
      for j in range(F // L):
        zbuf[r, pl.ds(j * L, L)] = zv
      return carry2
    lax.fori_loop(0, RPS // 2, zrow, 0)
    for q in range(2):
      pltpu.sync_copy(zbuf, acc.at[pl.ds(s * RPS + q * (RPS // 2), RPS // 2)])
    @pl.when(s == NS - 1)
    def _():
      pltpu.sync_copy(zbuf.at[pl.ds(0, RTAIL)], acc.at[pl.ds(TAIL0, RTAIL)])
    plsc.subcore_barrier()

    def scale(rows_v, k):
      def grp_body(g, carry2):
        wvec = w_a[k, pl.ds(g * L, L)]
        for r in range(L):
          wspl = wvec[jnp.full((L,), r, jnp.int32)]
          row = g * L + r
          for j in range(F // L):
            sl = (row, pl.ds(j * L, L))
            rows_v[sl] = rows_v[sl] * wspl
        return carry2
      lax.fori_loop(0, CH // L, grp_body, 0)

    def pipe_body(i, carry):
      for b in range(NB):
        k = NB * i + b
        nb = (b + NB - 1) % NB
        pltpu.make_async_copy(support_hbm.at[src_a.at[k]],
                              rows[b], semg[b]).wait()
        scale(rows[b], k)
        pltpu.async_copy(rows[b], acc.at[dst_a.at[k]], sems[b], add=True)
        if b == 0:
          @pl.when(i == 0)
          def _():
            pltpu.async_copy(support_hbm.at[src_a.at[NB - 1]],
                             rows[NB - 1], semg[NB - 1])
          @pl.when(i > 0)
          def _():
            pltpu.make_async_copy(rows[nb], acc.at[dst_a.at[k - 1]],
                                  sems[nb]).wait()
            pltpu.async_copy(support_hbm.at[src_a.at[k + NB - 1]],
                             rows[nb], semg[nb])
        else:
          @pl.when(k + NB - 1 < nchunk)
          def _():
            pltpu.make_async_copy(rows[nb], acc.at[dst_a.at[k - 1]],
                                  sems[nb]).wait()
            pltpu.async_copy(support_hbm.at[src_a.at[k + NB - 1]],
                             rows[nb], semg[nb])
      return carry

    lax.fori_loop(0, nchunk // NB, pipe_body, 0)
    for b in range(NB):
      pltpu.make_async_copy(rows[b], acc.at[dst_a.at[nchunk - NB + b]],
                            sems[b]).wait()
    plsc.subcore_barrier()
    pltpu.sync_copy(acc.at[pl.ds(s * RPS, RPS)],
                    out_hbm.at[c, pl.ds(s * RPS, RPS)])
    @pl.when(s == NS - 1)
    def _():
      pltpu.sync_copy(acc.at[pl.ds(TAIL0, RTAIL)],
                      out_hbm.at[c, pl.ds(TAIL0, RTAIL)])

  return spmm


def _mm(x, W):
  def body(x_ref, w_ref, o_ref):
    o_ref[...] = jnp.dot(x_ref[...], w_ref[...],
                         preferred_element_type=jnp.float32)
  return pl.pallas_call(
      body,
      out_shape=jax.ShapeDtypeStruct((x.shape[0], W.shape[1]), jnp.float32),
  )(x, W)


def _act(p, b):
  def body(p_ref, b_ref, o_ref):
    h = p_ref[0] + p_ref[1] + b_ref[...]
    o_ref[...] = jnp.where(h > 0, h, 0.01 * h)
  return pl.pallas_call(
      body,
      out_shape=jax.ShapeDtypeStruct((p.shape[1], p.shape[2]), jnp.float32),
  )(p, b.reshape(1, -1))


def _mm_act_mm(p, W2, b2, W3):
  def body(p_ref, w2_ref, b2_ref, w3_ref, o_ref):
    h = jnp.dot(p_ref[0] + p_ref[1], w2_ref[...],
                preferred_element_type=jnp.float32) + b2_ref[...]
    h = jnp.where(h > 0, h, 0.01 * h)
    o_ref[...] = jnp.dot(h, w3_ref[...], preferred_element_type=jnp.float32)
  return pl.pallas_call(
      body,
      out_shape=jax.ShapeDtypeStruct((p.shape[1], W3.shape[1]), jnp.float32),
  )(p, W2, b2.reshape(1, -1), W3)


def _final(p, b):
  def body(p_ref, b_ref, o_ref):
    z = p_ref[0] + b_ref[...]
    for i in range(1, p_ref.shape[0]):
      z = z + p_ref[i]
    m = jnp.max(z, axis=1, keepdims=True)
    e = jnp.exp(z - m)
    lse = jnp.log(jnp.sum(e, axis=1, keepdims=True)) + m
    o_ref[...] = z - lse
  return pl.pallas_call(
      body,
      out_shape=jax.ShapeDtypeStruct((p.shape[1], p.shape[2]), jnp.float32),
  )(p, b.reshape(1, -1))


def kernel(x, edge_index, edge_weight, W1, b1, W2, b2, W3, b3):
  pad = EPAD - E
  src = jnp.pad(edge_index[0].astype(jnp.int32), (0, pad)).reshape(-1, CH)
  dst = jnp.pad(edge_index[1].astype(jnp.int32), (0, pad)).reshape(-1, CH)
  w = jnp.pad(edge_weight, (0, pad)).reshape(-1, CH)

  W3p = jnp.pad(W3, ((0, 0), (0, 8)))
  b3p = jnp.concatenate([b3, jnp.full((8,), -1e30, jnp.float32)])

  s1 = _mm(x, W1)
  p1 = _make_spmm(16, 128, 32)(s1, src, dst, w)
  h1 = _act(p1, b1)
  p2 = _make_spmm(16, 112, 48)(h1, src, dst, w)
  s3 = _mm_act_mm(p2, W2, b2, W3p)
  p3 = _make_spmm(48, 128, 32)(s3, src, dst, w)
  return _final(p3, b3p)[:, :40]
